# Initial kernel scaffold; baseline (speedup 1.0000x reference)
import functools

import jax
import jax.numpy as jnp
from jax import lax
from jax.experimental import pallas as pl
from jax.experimental.pallas import tpu as pltpu
from jax.experimental.pallas import tpu_sc as plsc

NUM_SEG = 512
NUM_NODES = 100000
FEAT = 128
CHUNK = 80
NCHUNKS = NUM_NODES // CHUNK
NW = 32
BASE_PER_W = NCHUNKS // NW
EXTRA = NCHUNKS - BASE_PER_W * NW
SEG_PER_TILE = NUM_SEG // 16


def _seg_body(attr_hbm, batch_hbm, psum_hbm, pcnt_hbm,
              chunk_v, ids_v, ones_v, stage_v, cstage_v, acc_sh, cnt_sh):
    cid = lax.axis_index("c")
    sid = lax.axis_index("s")
    w = cid * 16 + sid

    zeros16 = jnp.zeros((16,), jnp.float32)
    ones16 = jnp.ones((16,), jnp.float32)
    for i in range(SEG_PER_TILE):
        for j in range(FEAT // 16):
            stage_v[i, pl.ds(j * 16, 16)] = zeros16
        cstage_v[i, pl.ds(0, 16)] = zeros16
    for i in range(CHUNK):
        ones_v[i, pl.ds(0, 16)] = ones16

    pltpu.sync_copy(stage_v, acc_sh.at[pl.ds(sid * SEG_PER_TILE, SEG_PER_TILE)])
    pltpu.sync_copy(cstage_v, cnt_sh.at[pl.ds(sid * SEG_PER_TILE, SEG_PER_TILE)])
    plsc.subcore_barrier()

    n_w = BASE_PER_W + jnp.where(w < EXTRA, 1, 0)
    start_w = BASE_PER_W * w + jnp.minimum(w, EXTRA)

    def body(i, carry):
        base = (start_w + i) * CHUNK
        pltpu.sync_copy(attr_hbm.at[pl.ds(base, CHUNK)], chunk_v)
        pltpu.sync_copy(batch_hbm.at[pl.ds(base, CHUNK)], ids_v)
        pltpu.sync_copy(chunk_v, acc_sh.at[ids_v], add=True)
        pltpu.sync_copy(ones_v, cnt_sh.at[ids_v], add=True)
        return carry

    lax.fori_loop(0, n_w, body, 0)
    plsc.subcore_barrier()

    row = sid * SEG_PER_TILE
    pltpu.sync_copy(acc_sh.at[pl.ds(row, SEG_PER_TILE)], stage_v)
    pltpu.sync_copy(stage_v, psum_hbm.at[pl.ds(cid * NUM_SEG + row, SEG_PER_TILE)])
    pltpu.sync_copy(cnt_sh.at[pl.ds(row, SEG_PER_TILE)], cstage_v)
    pltpu.sync_copy(cstage_v, pcnt_hbm.at[pl.ds(cid * NUM_SEG + row, SEG_PER_TILE)])


_seg_kernel = pl.kernel(
    _seg_body,
    out_type=[
        jax.ShapeDtypeStruct((2 * NUM_SEG, FEAT), jnp.float32),
        jax.ShapeDtypeStruct((2 * NUM_SEG, 16), jnp.float32),
    ],
    mesh=plsc.VectorSubcoreMesh(core_axis_name="c", subcore_axis_name="s"),
    scratch_types=[
        pltpu.VMEM((CHUNK, FEAT), jnp.float32),
        pltpu.VMEM((CHUNK,), jnp.int32),
        pltpu.VMEM((CHUNK, 16), jnp.float32),
        pltpu.VMEM((SEG_PER_TILE, FEAT), jnp.float32),
        pltpu.VMEM((SEG_PER_TILE, 16), jnp.float32),
        pltpu.VMEM_SHARED((NUM_SEG, FEAT), jnp.float32),
        pltpu.VMEM_SHARED((NUM_SEG, 16), jnp.float32),
    ],
)


def _combine_body(ps_ref, pc_ref, o_ref):
    s = ps_ref[0:NUM_SEG, :] + ps_ref[NUM_SEG:2 * NUM_SEG, :]
    c = pc_ref[0:NUM_SEG, 0:1] + pc_ref[NUM_SEG:2 * NUM_SEG, 0:1]
    o_ref[...] = s / jnp.maximum(c, 1.0)


def kernel(node_attr, batch):
    psum, pcnt = _seg_kernel(node_attr, batch)
    mean = pl.pallas_call(
        _combine_body,
        out_shape=jax.ShapeDtypeStruct((NUM_SEG, FEAT), jnp.float32),
    )(psum, pcnt)
    return mean.reshape(-1)


# speedup vs baseline: 3.9723x; 3.9723x over previous
import functools

import jax
import jax.numpy as jnp
from jax import lax
from jax.experimental import pallas as pl
from jax.experimental.pallas import tpu as pltpu
from jax.experimental.pallas import tpu_sc as plsc

NUM_SEG = 512
NUM_NODES = 100000
FEAT = 128
CHUNK = 80
NCHUNKS = NUM_NODES // CHUNK
NW = 32
BASE_PER_W = NCHUNKS // NW
EXTRA = NCHUNKS - BASE_PER_W * NW
SEG_PER_TILE = NUM_SEG // 16


def _seg_body(attr_hbm, batch_hbm, psum_hbm, pcnt_hbm,
              chunk_v, ids_v, ones_v, stage_v, cstage_v, acc_sh, cnt_sh):
    cid = lax.axis_index("c")
    sid = lax.axis_index("s")
    w = cid * 16 + sid

    zeros16 = jnp.zeros((16,), jnp.float32)
    ones16 = jnp.ones((16,), jnp.float32)
    for i in range(SEG_PER_TILE):
        for j in range(FEAT // 16):
            stage_v[i, pl.ds(j * 16, 16)] = zeros16
            cstage_v[i, pl.ds(j * 16, 16)] = zeros16
    for i in range(CHUNK):
        for j in range(FEAT // 16):
            ones_v[i, pl.ds(j * 16, 16)] = ones16

    pltpu.sync_copy(stage_v, acc_sh.at[pl.ds(sid * SEG_PER_TILE, SEG_PER_TILE)])
    pltpu.sync_copy(cstage_v, cnt_sh.at[pl.ds(sid * SEG_PER_TILE, SEG_PER_TILE)])
    plsc.subcore_barrier()

    n_w = BASE_PER_W + jnp.where(w < EXTRA, 1, 0)
    start_w = BASE_PER_W * w + jnp.minimum(w, EXTRA)

    def body(i, carry):
        base = (start_w + i) * CHUNK
        pltpu.sync_copy(attr_hbm.at[pl.ds(base, CHUNK)], chunk_v)
        pltpu.sync_copy(batch_hbm.at[pl.ds(base, CHUNK)], ids_v)
        pltpu.sync_copy(chunk_v, acc_sh.at[ids_v], add=True)
        pltpu.sync_copy(ones_v, cnt_sh.at[ids_v], add=True)
        return carry

    lax.fori_loop(0, n_w, body, 0)
    plsc.subcore_barrier()

    row = sid * SEG_PER_TILE
    pltpu.sync_copy(acc_sh.at[pl.ds(row, SEG_PER_TILE)], stage_v)
    pltpu.sync_copy(stage_v, psum_hbm.at[pl.ds(cid * NUM_SEG + row, SEG_PER_TILE)])
    pltpu.sync_copy(cnt_sh.at[pl.ds(row, SEG_PER_TILE)], cstage_v)
    pltpu.sync_copy(cstage_v, pcnt_hbm.at[pl.ds(cid * NUM_SEG + row, SEG_PER_TILE)])


_seg_kernel = pl.kernel(
    _seg_body,
    out_type=[
        jax.ShapeDtypeStruct((2 * NUM_SEG, FEAT), jnp.float32),
        jax.ShapeDtypeStruct((2 * NUM_SEG, FEAT), jnp.float32),
    ],
    mesh=plsc.VectorSubcoreMesh(core_axis_name="c", subcore_axis_name="s"),
    scratch_types=[
        pltpu.VMEM((CHUNK, FEAT), jnp.float32),
        pltpu.VMEM((CHUNK,), jnp.int32),
        pltpu.VMEM((CHUNK, FEAT), jnp.float32),
        pltpu.VMEM((SEG_PER_TILE, FEAT), jnp.float32),
        pltpu.VMEM((SEG_PER_TILE, FEAT), jnp.float32),
        pltpu.VMEM_SHARED((NUM_SEG, FEAT), jnp.float32),
        pltpu.VMEM_SHARED((NUM_SEG, FEAT), jnp.float32),
    ],
)


def _combine_body(ps_ref, pc_ref, o_ref):
    s = ps_ref[0:NUM_SEG, :] + ps_ref[NUM_SEG:2 * NUM_SEG, :]
    c = pc_ref[0:NUM_SEG, 0:1] + pc_ref[NUM_SEG:2 * NUM_SEG, 0:1]
    o_ref[...] = s / jnp.maximum(c, 1.0)


def kernel(node_attr, batch):
    psum, pcnt = _seg_kernel(node_attr, batch)
    mean = pl.pallas_call(
        _combine_body,
        out_shape=jax.ShapeDtypeStruct((NUM_SEG, FEAT), jnp.float32),
    )(psum, pcnt)
    return mean.reshape(-1)


# speedup vs baseline: 5.1908x; 1.3067x over previous
import jax
import jax.numpy as jnp
from jax import lax
from jax.experimental import pallas as pl
from jax.experimental.pallas import tpu as pltpu
from jax.experimental.pallas import tpu_sc as plsc

NUM_SEG = 512
NUM_NODES = 100000
FEAT = 128
CHUNK = 80
NCHUNKS = NUM_NODES // CHUNK
NW = 32
BASE_PER_W = NCHUNKS // NW
EXTRA = NCHUNKS - BASE_PER_W * NW
STEPS = (BASE_PER_W + 2) // 2
SEG_PER_TILE = NUM_SEG // 16


def _seg_body(attr_hbm, batch_hbm, psum_hbm, pcnt_hbm,
              chunk0, chunk1, ids0, ids1, ones_v, stage_v,
              acc_sh, cnt_sh, sg0, sg1, ss0, ss1):
    cid = lax.axis_index("c")
    sid = lax.axis_index("s")
    w = cid * 16 + sid

    zeros16 = jnp.zeros((16,), jnp.float32)
    ones16 = jnp.ones((16,), jnp.float32)
    for i in range(SEG_PER_TILE):
        for j in range(FEAT // 16):
            stage_v[i, pl.ds(j * 16, 16)] = zeros16
    for i in range(CHUNK):
        for j in range(FEAT // 16):
            ones_v[i, pl.ds(j * 16, 16)] = ones16

    pltpu.sync_copy(stage_v, acc_sh.at[pl.ds(sid * SEG_PER_TILE, SEG_PER_TILE)])
    pltpu.sync_copy(stage_v, cnt_sh.at[pl.ds(sid * SEG_PER_TILE, SEG_PER_TILE)])
    plsc.subcore_barrier()

    n_w = BASE_PER_W + jnp.where(w < EXTRA, 1, 0)
    start_w = BASE_PER_W * w + jnp.minimum(w, EXTRA)

    def gather(i, chunk_v, ids_v, sem):
        base = (start_w + i) * CHUNK
        pltpu.async_copy(attr_hbm.at[pl.ds(base, CHUNK)], chunk_v, sem)
        pltpu.async_copy(batch_hbm.at[pl.ds(base, CHUNK)], ids_v, sem)

    def gather_wait(chunk_v, ids_v, sem):
        pltpu.make_async_copy(attr_hbm.at[pl.ds(0, CHUNK)], chunk_v, sem).wait()
        pltpu.make_async_copy(batch_hbm.at[pl.ds(0, CHUNK)], ids_v, sem).wait()

    def scatter(chunk_v, ids_v, sem):
        pltpu.async_copy(chunk_v, acc_sh.at[ids_v], sem, add=True)
        pltpu.async_copy(ones_v, cnt_sh.at[ids_v], sem, add=True)

    def scatter_wait(chunk_v, sem):
        pltpu.make_async_copy(chunk_v, acc_sh.at[pl.ds(0, CHUNK)], sem).wait()
        pltpu.make_async_copy(ones_v, cnt_sh.at[pl.ds(0, CHUNK)], sem).wait()

    gather(0, chunk0, ids0, sg0)
    gather(1, chunk1, ids1, sg1)
    for j in range(STEPS):
        a = 2 * j
        b = a + 1
        gather_wait(chunk0, ids0, sg0)
        scatter(chunk0, ids0, ss0)
        scatter_wait(chunk0, ss0)

        @pl.when(a + 2 < n_w)
        def _():
            gather(a + 2, chunk0, ids0, sg0)

        @pl.when(b < n_w)
        def _():
            gather_wait(chunk1, ids1, sg1)
            scatter(chunk1, ids1, ss1)
            scatter_wait(chunk1, ss1)

        @pl.when(b + 2 < n_w)
        def _():
            gather(b + 2, chunk1, ids1, sg1)

    plsc.subcore_barrier()

    row = sid * SEG_PER_TILE
    pltpu.sync_copy(acc_sh.at[pl.ds(row, SEG_PER_TILE)], stage_v)
    pltpu.sync_copy(stage_v, psum_hbm.at[pl.ds(cid * NUM_SEG + row, SEG_PER_TILE)])
    pltpu.sync_copy(cnt_sh.at[pl.ds(row, SEG_PER_TILE)], stage_v)
    pltpu.sync_copy(stage_v, pcnt_hbm.at[pl.ds(cid * NUM_SEG + row, SEG_PER_TILE)])


_seg_kernel = pl.kernel(
    _seg_body,
    out_type=[
        jax.ShapeDtypeStruct((2 * NUM_SEG, FEAT), jnp.float32),
        jax.ShapeDtypeStruct((2 * NUM_SEG, FEAT), jnp.float32),
    ],
    mesh=plsc.VectorSubcoreMesh(core_axis_name="c", subcore_axis_name="s"),
    scratch_types=[
        pltpu.VMEM((CHUNK, FEAT), jnp.float32),
        pltpu.VMEM((CHUNK, FEAT), jnp.float32),
        pltpu.VMEM((CHUNK,), jnp.int32),
        pltpu.VMEM((CHUNK,), jnp.int32),
        pltpu.VMEM((CHUNK, FEAT), jnp.float32),
        pltpu.VMEM((SEG_PER_TILE, FEAT), jnp.float32),
        pltpu.VMEM_SHARED((NUM_SEG, FEAT), jnp.float32),
        pltpu.VMEM_SHARED((NUM_SEG, FEAT), jnp.float32),
        pltpu.SemaphoreType.DMA,
        pltpu.SemaphoreType.DMA,
        pltpu.SemaphoreType.DMA,
        pltpu.SemaphoreType.DMA,
    ],
)


def _combine_body(ps_ref, pc_ref, o_ref):
    s = ps_ref[0:NUM_SEG, :] + ps_ref[NUM_SEG:2 * NUM_SEG, :]
    c = pc_ref[0:NUM_SEG, :] + pc_ref[NUM_SEG:2 * NUM_SEG, :]
    o_ref[...] = s / jnp.maximum(c, 1.0)


def kernel(node_attr, batch):
    psum, pcnt = _seg_kernel(node_attr, batch)
    mean = pl.pallas_call(
        _combine_body,
        out_shape=jax.ShapeDtypeStruct((NUM_SEG, FEAT), jnp.float32),
    )(psum, pcnt)
    return mean.reshape(-1)


# speedup vs baseline: 7.4750x; 1.4401x over previous
import jax
import jax.numpy as jnp
from jax import lax
from jax.experimental import pallas as pl
from jax.experimental.pallas import tpu as pltpu
from jax.experimental.pallas import tpu_sc as plsc

NUM_SEG = 512
NUM_NODES = 100000
FEAT = 128
CHUNK = 80
NCHUNKS = NUM_NODES // CHUNK
NW = 32
BASE_PER_W = NCHUNKS // NW
EXTRA = NCHUNKS - BASE_PER_W * NW
STEPS = (BASE_PER_W + 2) // 2
SEG_PER_TILE = NUM_SEG // 16


def _seg_body(attr_hbm, batch_hbm, psum_hbm, pcnt_hbm,
              chunk0, chunk1, ids0, ids1, ones_v, stage_v, cstage_v,
              acc_sh, cnt_sh, sg0, sg1, ss0, ss1):
    cid = lax.axis_index("c")
    sid = lax.axis_index("s")
    w = cid * 16 + sid

    zeros16 = jnp.zeros((16,), jnp.float32)
    ones16 = jnp.ones((16,), jnp.float32)
    for i in range(SEG_PER_TILE):
        for j in range(FEAT // 16):
            stage_v[i, pl.ds(j * 16, 16)] = zeros16
    for i in range(SEG_PER_TILE // 16):
        cstage_v[pl.ds(i * 16, 16)] = zeros16
    for i in range(CHUNK // 16):
        ones_v[pl.ds(i * 16, 16)] = ones16

    pltpu.sync_copy(stage_v, acc_sh.at[pl.ds(sid * SEG_PER_TILE, SEG_PER_TILE)])
    pltpu.sync_copy(cstage_v, cnt_sh.at[pl.ds(sid * SEG_PER_TILE, SEG_PER_TILE)])
    plsc.subcore_barrier()

    n_w = BASE_PER_W + jnp.where(w < EXTRA, 1, 0)
    start_w = BASE_PER_W * w + jnp.minimum(w, EXTRA)

    def gather(i, chunk_v, ids_v, sem):
        base = (start_w + i) * CHUNK
        pltpu.async_copy(attr_hbm.at[pl.ds(base, CHUNK)], chunk_v, sem)
        pltpu.async_copy(batch_hbm.at[pl.ds(base, CHUNK)], ids_v, sem)

    def gather_wait(chunk_v, ids_v, sem):
        pltpu.make_async_copy(attr_hbm.at[pl.ds(0, CHUNK)], chunk_v, sem).wait()
        pltpu.make_async_copy(batch_hbm.at[pl.ds(0, CHUNK)], ids_v, sem).wait()

    def scatter(chunk_v, ids_v, sem):
        pltpu.async_copy(chunk_v, acc_sh.at[ids_v], sem, add=True)
        pltpu.async_copy(ones_v, cnt_sh.at[ids_v], sem, add=True)

    def scatter_wait(chunk_v, sem):
        pltpu.make_async_copy(chunk_v, acc_sh.at[pl.ds(0, CHUNK)], sem).wait()
        pltpu.make_async_copy(ones_v, cnt_sh.at[pl.ds(0, CHUNK)], sem).wait()

    gather(0, chunk0, ids0, sg0)
    gather(1, chunk1, ids1, sg1)
    for j in range(STEPS):
        a = 2 * j
        b = a + 1
        gather_wait(chunk0, ids0, sg0)
        scatter(chunk0, ids0, ss0)
        scatter_wait(chunk0, ss0)

        @pl.when(a + 2 < n_w)
        def _():
            gather(a + 2, chunk0, ids0, sg0)

        @pl.when(b < n_w)
        def _():
            gather_wait(chunk1, ids1, sg1)
            scatter(chunk1, ids1, ss1)
            scatter_wait(chunk1, ss1)

        @pl.when(b + 2 < n_w)
        def _():
            gather(b + 2, chunk1, ids1, sg1)

    plsc.subcore_barrier()

    row = sid * SEG_PER_TILE
    pltpu.sync_copy(acc_sh.at[pl.ds(row, SEG_PER_TILE)], stage_v)
    pltpu.sync_copy(stage_v, psum_hbm.at[pl.ds(cid * NUM_SEG + row, SEG_PER_TILE)])
    pltpu.sync_copy(cnt_sh.at[pl.ds(row, SEG_PER_TILE)], cstage_v)
    pltpu.sync_copy(cstage_v, pcnt_hbm.at[cid, pl.ds(row, SEG_PER_TILE)])


_seg_kernel = pl.kernel(
    _seg_body,
    out_type=[
        jax.ShapeDtypeStruct((2 * NUM_SEG, FEAT), jnp.float32),
        jax.ShapeDtypeStruct((16, NUM_SEG), jnp.float32),
    ],
    mesh=plsc.VectorSubcoreMesh(core_axis_name="c", subcore_axis_name="s"),
    scratch_types=[
        pltpu.VMEM((CHUNK, FEAT), jnp.float32),
        pltpu.VMEM((CHUNK, FEAT), jnp.float32),
        pltpu.VMEM((CHUNK,), jnp.int32),
        pltpu.VMEM((CHUNK,), jnp.int32),
        pltpu.VMEM((CHUNK,), jnp.float32),
        pltpu.VMEM((SEG_PER_TILE, FEAT), jnp.float32),
        pltpu.VMEM((SEG_PER_TILE,), jnp.float32),
        pltpu.VMEM_SHARED((NUM_SEG, FEAT), jnp.float32),
        pltpu.VMEM_SHARED((NUM_SEG,), jnp.float32),
        pltpu.SemaphoreType.DMA,
        pltpu.SemaphoreType.DMA,
        pltpu.SemaphoreType.DMA,
        pltpu.SemaphoreType.DMA,
    ],
)


def _combine_body(ps_ref, pc_ref, o_ref):
    s = ps_ref[0:NUM_SEG, :] + ps_ref[NUM_SEG:2 * NUM_SEG, :]
    ct = jnp.transpose(pc_ref[...], (1, 0))
    c = ct[:, 0:1] + ct[:, 1:2]
    o_ref[...] = s / jnp.maximum(c, 1.0)


def kernel(node_attr, batch):
    psum, pcnt = _seg_kernel(node_attr, batch)
    mean = pl.pallas_call(
        _combine_body,
        out_shape=jax.ShapeDtypeStruct((NUM_SEG, FEAT), jnp.float32),
    )(psum, pcnt)
    return mean.reshape(-1)
